# Initial kernel scaffold; baseline (speedup 1.0000x reference)
import jax
import jax.numpy as jnp
from jax.experimental import pallas as pl


def kernel(feat1, feat2, adj_spatial1, adj_spatial2, e1w1, e1w2, e2w1, e2w2, d1w, d2w):
    raise NotImplementedError("write your pallas kernel here")


# speedup vs baseline: 1.2013x; 1.2013x over previous
import functools

import jax
import jax.numpy as jnp
from jax.experimental import pallas as pl

N = 4096
HID = 64
O = 128
THRESH = 0.6
BLK = 256


def _mm_kernel(a_ref, b_ref, o_ref):
    o_ref[...] = jnp.dot(a_ref[...], b_ref[...],
                         preferred_element_type=jnp.float32)


def _mm(a, b, blk=BLK):
    m, k = a.shape
    _, n = b.shape
    return pl.pallas_call(
        _mm_kernel,
        grid=(m // blk,),
        in_specs=[
            pl.BlockSpec((blk, k), lambda i: (i, 0)),
            pl.BlockSpec((k, n), lambda i: (0, 0)),
        ],
        out_specs=pl.BlockSpec((blk, n), lambda i: (i, 0)),
        out_shape=jax.ShapeDtypeStruct((m, n), jnp.float32),
    )(a, b)


def _pre_kernel(adj_ref, u_ref, w2_ref, hn_ref, yin_ref):
    h = jnp.dot(adj_ref[...], u_ref[...], preferred_element_type=jnp.float32)
    h = jnp.maximum(h, 0.0)
    norm = jnp.sqrt(jnp.sum(h * h, axis=1, keepdims=True))
    hn_ref[...] = h / jnp.maximum(norm, 1e-12)
    yin_ref[...] = jnp.dot(h, w2_ref[...], preferred_element_type=jnp.float32)


def _pre(adj, u, w2):
    return pl.pallas_call(
        _pre_kernel,
        grid=(N // BLK,),
        in_specs=[
            pl.BlockSpec((BLK, N), lambda i: (i, 0)),
            pl.BlockSpec((N, HID), lambda i: (0, 0)),
            pl.BlockSpec((HID, O), lambda i: (0, 0)),
        ],
        out_specs=[
            pl.BlockSpec((BLK, HID), lambda i: (i, 0)),
            pl.BlockSpec((BLK, O), lambda i: (i, 0)),
        ],
        out_shape=[
            jax.ShapeDtypeStruct((N, HID), jnp.float32),
            jax.ShapeDtypeStruct((N, O), jnp.float32),
        ],
    )(adj, u, w2)


def _simagg_kernel(hnb_ref, hn_ref, yin_ref, y_ref):
    s = jax.lax.dot_general(
        hnb_ref[...], hn_ref[...],
        dimension_numbers=(((1,), (1,)), ((), ())),
        preferred_element_type=jnp.float32)
    s = jnp.where(s < THRESH, 0.0, s)
    rs = jnp.sum(s, axis=1, keepdims=True)
    agg = jnp.dot(s, yin_ref[...], preferred_element_type=jnp.float32)
    y_ref[...] = agg / jnp.maximum(rs, 1e-12)


def _simagg(hn, yin):
    return pl.pallas_call(
        _simagg_kernel,
        grid=(N // BLK,),
        in_specs=[
            pl.BlockSpec((BLK, HID), lambda i: (i, 0)),
            pl.BlockSpec((N, HID), lambda i: (0, 0)),
            pl.BlockSpec((N, O), lambda i: (0, 0)),
        ],
        out_specs=pl.BlockSpec((BLK, O), lambda i: (i, 0)),
        out_shape=jax.ShapeDtypeStruct((N, O), jnp.float32),
    )(hn, hn, yin)


def _xz_kernel(y1_ref, y2_ref, d1_ref, d2_ref, x1_ref, x2_ref, z_ref):
    y1 = y1_ref[...]
    y2 = y2_ref[...]
    x1_ref[...] = jnp.dot(y1, d1_ref[...], preferred_element_type=jnp.float32)
    x2_ref[...] = jnp.dot(y2, d2_ref[...], preferred_element_type=jnp.float32)
    z_ref[...] = (y1 + y2) * 0.5


def _xz(y1, y2, d1w, d2w):
    d1o = d1w.shape[1]
    d2o = d2w.shape[1]
    return pl.pallas_call(
        _xz_kernel,
        grid=(N // BLK,),
        in_specs=[
            pl.BlockSpec((BLK, O), lambda i: (i, 0)),
            pl.BlockSpec((BLK, O), lambda i: (i, 0)),
            pl.BlockSpec((O, d1o), lambda i: (0, 0)),
            pl.BlockSpec((O, d2o), lambda i: (0, 0)),
        ],
        out_specs=[
            pl.BlockSpec((BLK, d1o), lambda i: (i, 0)),
            pl.BlockSpec((BLK, d2o), lambda i: (i, 0)),
            pl.BlockSpec((BLK, O), lambda i: (i, 0)),
        ],
        out_shape=[
            jax.ShapeDtypeStruct((N, d1o), jnp.float32),
            jax.ShapeDtypeStruct((N, d2o), jnp.float32),
            jax.ShapeDtypeStruct((N, O), jnp.float32),
        ],
    )(y1, y2, d1w, d2w)


def kernel(feat1, feat2, adj_spatial1, adj_spatial2,
           e1w1, e1w2, e2w1, e2w2, d1w, d2w):
    u1 = _mm(feat1, e1w1)
    u2 = _mm(feat2, e2w1)
    hn1, yin1 = _pre(adj_spatial1, u1, e1w2)
    hn2, yin2 = _pre(adj_spatial2, u2, e2w2)
    y1 = _simagg(hn1, yin1)
    y2 = _simagg(hn2, yin2)
    x1, x2, z = _xz(y1, y2, d1w, d2w)
    recon1 = _mm(adj_spatial1, x1)
    recon2 = _mm(adj_spatial2, x2)
    return (y1, y2, z, recon1, recon2)
